# baseline (device time: 84119 ns/iter reference)
import jax
import jax.numpy as jnp
from jax import lax
from jax.experimental import pallas as pl
from jax.experimental.pallas import tpu as pltpu

N_DEV = 16
N_EXPERTS = 32
CAPACITY = 102
N_TOK = 256
D_IN = 128
D_OUT = 256
E_PER = 2
W_ROWS = E_PER * D_IN
PAY_ROWS = W_ROWS + 8


def kernel(x, router_W, route_idx, expert_W):
    del router_W

    onehot = (route_idx == jnp.arange(N_EXPERTS, dtype=jnp.int32)[None, :])
    cums = jnp.cumsum(onehot.astype(jnp.int32), axis=0)
    local_rank = jnp.take_along_axis(cums, route_idx, axis=1) - 1
    counts_local = cums[-1]

    w_flat = expert_W.reshape(W_ROWS, D_OUT)
    pad = jnp.zeros((8, D_OUT), jnp.int32)
    pad = pad.at[0, :N_EXPERTS].set(counts_local)
    payload = jnp.concatenate(
        [w_flat, lax.bitcast_convert_type(pad, jnp.float32)], axis=0
    )

    def body(payload_ref, x_ref, ri_ref, rank_ref, out_ref,
             comm_ref, send_sems, recv_sems):
        my = lax.axis_index("i")
        left = lax.rem(my - 1 + N_DEV, N_DEV)
        right = lax.rem(my + 1, N_DEV)

        barrier_sem = pltpu.get_barrier_semaphore()
        for nbr in (left, right):
            pl.semaphore_signal(
                barrier_sem, inc=1,
                device_id=(nbr,), device_id_type=pl.DeviceIdType.MESH,
            )
        pl.semaphore_wait(barrier_sem, 2)

        comm_ref[0] = payload_ref[:, :]

        xv = x_ref[:, :]
        ri = ri_ref[:, :]

        def process(k, base):
            src = lax.rem(my - k + N_DEV, N_DEV)
            e0 = E_PER * src
            m0 = (ri == e0).astype(jnp.float32)
            m1 = (ri == e0 + 1).astype(jnp.float32)
            w0 = comm_ref[k, 0:D_IN, :]
            w1 = comm_ref[k, D_IN:W_ROWS, :]
            contrib = (
                jnp.dot(xv * m0, w0, preferred_element_type=jnp.float32)
                + jnp.dot(xv * m1, w1, preferred_element_type=jnp.float32)
            )
            if k == 0:
                out_ref[:, :] = contrib
            else:
                out_ref[:, :] = out_ref[:, :] + contrib
            counts_k = lax.bitcast_convert_type(
                comm_ref[k, W_ROWS:W_ROWS + 1, :], jnp.int32
            )
            pred = jnp.logical_and(k >= 1, k <= my)
            return base + jnp.where(pred, counts_k, 0)

        base = jnp.zeros((1, D_OUT), jnp.int32)
        base = process(0, base)

        for h in range(N_DEV - 1):
            rdma = pltpu.make_async_remote_copy(
                src_ref=comm_ref.at[h],
                dst_ref=comm_ref.at[h + 1],
                send_sem=send_sems.at[h],
                recv_sem=recv_sems.at[h],
                device_id=(right,),
                device_id_type=pl.DeviceIdType.MESH,
            )
            rdma.start()
            rdma.wait()
            base = process(h + 1, base)

        col = lax.broadcasted_iota(jnp.int32, (N_TOK, D_OUT), 1)
        oh = (ri == col)
        base_i = jnp.sum(
            jnp.where(oh, jnp.broadcast_to(base, (N_TOK, D_OUT)), 0),
            axis=1, keepdims=True,
        )
        keep = ((base_i + rank_ref[:, :]) < CAPACITY).astype(jnp.float32)
        out_ref[:, :] = out_ref[:, :] * keep

    return pl.pallas_call(
        body,
        out_shape=jax.ShapeDtypeStruct((N_TOK, D_OUT), jnp.float32),
        in_specs=[
            pl.BlockSpec(memory_space=pltpu.VMEM),
            pl.BlockSpec(memory_space=pltpu.VMEM),
            pl.BlockSpec(memory_space=pltpu.VMEM),
            pl.BlockSpec(memory_space=pltpu.VMEM),
        ],
        out_specs=pl.BlockSpec(memory_space=pltpu.VMEM),
        scratch_shapes=[
            pltpu.VMEM((N_DEV, PAY_ROWS, D_OUT), jnp.float32),
            pltpu.SemaphoreType.DMA((N_DEV - 1,)),
            pltpu.SemaphoreType.DMA((N_DEV - 1,)),
        ],
        compiler_params=pltpu.CompilerParams(collective_id=0),
    )(payload, x, route_idx, local_rank)


# device time: 50101 ns/iter; 1.6790x vs baseline; 1.6790x over previous
import jax
import jax.numpy as jnp
from jax import lax
from jax.experimental import pallas as pl
from jax.experimental.pallas import tpu as pltpu

N_DEV = 16
N_EXPERTS = 32
CAPACITY = 102
N_TOK = 256
D_IN = 128
D_OUT = 256
E_PER = 2
W_ROWS = E_PER * D_IN
PAY_ROWS = W_ROWS + 8


def kernel(x, router_W, route_idx, expert_W):
    del router_W

    onehot = (route_idx == jnp.arange(N_EXPERTS, dtype=jnp.int32)[None, :])
    cums = jnp.cumsum(onehot.astype(jnp.int32), axis=0)
    local_rank = jnp.take_along_axis(cums, route_idx, axis=1) - 1
    counts_local = cums[-1]

    w_flat = expert_W.reshape(W_ROWS, D_OUT)
    pad = jnp.zeros((8, D_OUT), jnp.int32)
    pad = pad.at[0, :N_EXPERTS].set(counts_local)
    payload = jnp.concatenate(
        [w_flat, lax.bitcast_convert_type(pad, jnp.float32)], axis=0
    )

    R_HOPS = N_DEV // 2
    L_HOPS = N_DEV - 1 - R_HOPS

    def body(payload_ref, x_ref, ri_ref, rank_ref, out_ref,
             comm_ref, send_r, recv_r, send_l, recv_l):
        my = lax.axis_index("i")
        left = lax.rem(my - 1 + N_DEV, N_DEV)
        right = lax.rem(my + 1, N_DEV)

        barrier_sem = pltpu.get_barrier_semaphore()
        for nbr in (left, right):
            pl.semaphore_signal(
                barrier_sem, inc=1,
                device_id=(nbr,), device_id_type=pl.DeviceIdType.MESH,
            )
        pl.semaphore_wait(barrier_sem, 2)

        comm_ref[0] = payload_ref[:, :]

        xv = x_ref[:, :]
        ri = ri_ref[:, :]

        def process(slot, src, base):
            e0 = E_PER * src
            m0 = (ri == e0).astype(jnp.float32)
            m1 = (ri == e0 + 1).astype(jnp.float32)
            w0 = comm_ref[slot, 0:D_IN, :]
            w1 = comm_ref[slot, D_IN:W_ROWS, :]
            contrib = (
                jnp.dot(xv * m0, w0, preferred_element_type=jnp.float32)
                + jnp.dot(xv * m1, w1, preferred_element_type=jnp.float32)
            )
            if slot == 0:
                out_ref[:, :] = contrib
            else:
                out_ref[:, :] = out_ref[:, :] + contrib
            counts_k = lax.bitcast_convert_type(
                comm_ref[slot, W_ROWS:W_ROWS + 1, :], jnp.int32
            )
            return base + jnp.where(src < my, counts_k, 0)

        def send(src_slot, dst_slot, sem_arr, sem_idx, rsem_arr, nbr):
            rdma = pltpu.make_async_remote_copy(
                src_ref=comm_ref.at[src_slot],
                dst_ref=comm_ref.at[dst_slot],
                send_sem=sem_arr.at[sem_idx],
                recv_sem=rsem_arr.at[sem_idx],
                device_id=(nbr,),
                device_id_type=pl.DeviceIdType.MESH,
            )
            rdma.start()
            return rdma

        def r_slot(j):
            return j

        def l_slot(j):
            return R_HOPS + j

        rdmas = []
        rdmas.append(send(0, r_slot(1), send_r, 0, recv_r, right))
        rdmas.append(send(0, l_slot(1), send_l, 0, recv_l, left))

        base = jnp.zeros((1, D_OUT), jnp.int32)
        base = process(0, my, base)

        for j in range(1, R_HOPS + 1):
            rdmas[2 * (j - 1)].wait_recv()
            if j < R_HOPS:
                rdmas.append(
                    send(r_slot(j), r_slot(j + 1), send_r, j, recv_r, right)
                )
            if j <= L_HOPS:
                rdmas[2 * (j - 1) + 1].wait_recv()
                if j < L_HOPS:
                    rdmas.append(
                        send(l_slot(j), l_slot(j + 1), send_l, j, recv_l, left)
                    )
            base = process(r_slot(j), lax.rem(my - j + N_DEV, N_DEV), base)
            if j <= L_HOPS:
                base = process(l_slot(j), lax.rem(my + j, N_DEV), base)

        for rdma in rdmas:
            rdma.wait_send()

        col = lax.broadcasted_iota(jnp.int32, (N_TOK, D_OUT), 1)
        oh = (ri == col)
        base_i = jnp.sum(
            jnp.where(oh, jnp.broadcast_to(base, (N_TOK, D_OUT)), 0),
            axis=1, keepdims=True,
        )
        keep = ((base_i + rank_ref[:, :]) < CAPACITY).astype(jnp.float32)
        out_ref[:, :] = out_ref[:, :] * keep

    return pl.pallas_call(
        body,
        out_shape=jax.ShapeDtypeStruct((N_TOK, D_OUT), jnp.float32),
        in_specs=[
            pl.BlockSpec(memory_space=pltpu.VMEM),
            pl.BlockSpec(memory_space=pltpu.VMEM),
            pl.BlockSpec(memory_space=pltpu.VMEM),
            pl.BlockSpec(memory_space=pltpu.VMEM),
        ],
        out_specs=pl.BlockSpec(memory_space=pltpu.VMEM),
        scratch_shapes=[
            pltpu.VMEM((N_DEV, PAY_ROWS, D_OUT), jnp.float32),
            pltpu.SemaphoreType.DMA((R_HOPS,)),
            pltpu.SemaphoreType.DMA((R_HOPS,)),
            pltpu.SemaphoreType.DMA((L_HOPS,)),
            pltpu.SemaphoreType.DMA((L_HOPS,)),
        ],
        compiler_params=pltpu.CompilerParams(collective_id=0),
    )(payload, x, route_idx, local_rank)


# device time: 44969 ns/iter; 1.8706x vs baseline; 1.1141x over previous
import jax
import jax.numpy as jnp
from jax import lax
from jax.experimental import pallas as pl
from jax.experimental.pallas import tpu as pltpu

N_DEV = 16
N_EXPERTS = 32
CAPACITY = 102
N_TOK = 256
D_IN = 128
D_OUT = 256
E_PER = 2
W_ROWS = E_PER * D_IN
PAY_ROWS = W_ROWS + 8


def kernel(x, router_W, route_idx, expert_W):
    del router_W

    onehot = (route_idx == jnp.arange(N_EXPERTS, dtype=jnp.int32)[None, :])
    cums = jnp.cumsum(onehot.astype(jnp.int32), axis=0)
    local_rank = jnp.take_along_axis(cums, route_idx, axis=1) - 1
    counts_local = cums[-1]

    w_flat = expert_W.reshape(W_ROWS, D_OUT)
    pad = jnp.zeros((8, D_OUT), jnp.int32)
    pad = pad.at[0, :N_EXPERTS].set(counts_local)
    payload = jnp.concatenate(
        [w_flat, lax.bitcast_convert_type(pad, jnp.float32)], axis=0
    )

    R_HOPS = N_DEV // 2
    L_HOPS = N_DEV - 1 - R_HOPS

    def body(payload_ref, x_ref, ri_ref, rank_ref, out_ref,
             comm_ref, send_r, recv_r, send_l, recv_l):
        my = lax.axis_index("i")
        left = lax.rem(my - 1 + N_DEV, N_DEV)
        right = lax.rem(my + 1, N_DEV)

        barrier_sem = pltpu.get_barrier_semaphore()
        for nbr in (left, right):
            pl.semaphore_signal(
                barrier_sem, inc=1,
                device_id=(nbr,), device_id_type=pl.DeviceIdType.MESH,
            )
        pl.semaphore_wait(barrier_sem, 2)

        comm_ref[0] = payload_ref[:, :]

        xv = x_ref[:, :]
        ri = ri_ref[:, :]

        def process(slot, src, base):
            e0 = E_PER * src
            m0 = (ri == e0).astype(jnp.float32)
            m1 = (ri == e0 + 1).astype(jnp.float32)
            w0 = comm_ref[slot, 0:D_IN, :]
            w1 = comm_ref[slot, D_IN:W_ROWS, :]
            contrib = (
                jnp.dot(xv * m0, w0, preferred_element_type=jnp.float32)
                + jnp.dot(xv * m1, w1, preferred_element_type=jnp.float32)
            )
            if slot == 0:
                out_ref[:, :] = contrib
            else:
                out_ref[:, :] = out_ref[:, :] + contrib
            counts_k = lax.bitcast_convert_type(
                comm_ref[slot, W_ROWS:W_ROWS + 1, :], jnp.int32
            )
            return base + jnp.where(src < my, counts_k, 0)

        SUBS = ((0, D_IN), (D_IN, PAY_ROWS - D_IN))

        def send_sub(src_slot, dst_slot, sem_arr, j, rsem_arr, nbr, c):
            off, sz = SUBS[c]
            rdma = pltpu.make_async_remote_copy(
                src_ref=comm_ref.at[src_slot, pl.ds(off, sz)],
                dst_ref=comm_ref.at[dst_slot, pl.ds(off, sz)],
                send_sem=sem_arr.at[j, c],
                recv_sem=rsem_arr.at[j, c],
                device_id=(nbr,),
                device_id_type=pl.DeviceIdType.MESH,
            )
            rdma.start()
            return rdma

        def send(src_slot, dst_slot, sem_arr, j, rsem_arr, nbr):
            return [
                send_sub(src_slot, dst_slot, sem_arr, j, rsem_arr, nbr, c)
                for c in range(len(SUBS))
            ]

        def r_slot(j):
            return j

        def l_slot(j):
            return R_HOPS + j

        all_rdmas = []
        r_prev = send(0, r_slot(1), send_r, 0, recv_r, right)
        l_prev = send(0, l_slot(1), send_l, 0, recv_l, left)
        all_rdmas += r_prev + l_prev

        base = jnp.zeros((1, D_OUT), jnp.int32)
        base = process(0, my, base)

        for j in range(1, R_HOPS + 1):
            r_next, l_next = [], []
            for c in range(len(SUBS)):
                r_prev[c].wait_recv()
                if j < R_HOPS:
                    r_next.append(
                        send_sub(r_slot(j), r_slot(j + 1),
                                 send_r, j, recv_r, right, c)
                    )
                if j <= L_HOPS:
                    l_prev[c].wait_recv()
                    if j < L_HOPS:
                        l_next.append(
                            send_sub(l_slot(j), l_slot(j + 1),
                                     send_l, j, recv_l, left, c)
                        )
            all_rdmas += r_next + l_next
            base = process(r_slot(j), lax.rem(my - j + N_DEV, N_DEV), base)
            if j <= L_HOPS:
                base = process(l_slot(j), lax.rem(my + j, N_DEV), base)
            r_prev, l_prev = r_next, l_next

        for rdma in all_rdmas:
            rdma.wait_send()

        col = lax.broadcasted_iota(jnp.int32, (N_TOK, D_OUT), 1)
        oh = (ri == col)
        base_i = jnp.sum(
            jnp.where(oh, jnp.broadcast_to(base, (N_TOK, D_OUT)), 0),
            axis=1, keepdims=True,
        )
        keep = ((base_i + rank_ref[:, :]) < CAPACITY).astype(jnp.float32)
        out_ref[:, :] = out_ref[:, :] * keep

    return pl.pallas_call(
        body,
        out_shape=jax.ShapeDtypeStruct((N_TOK, D_OUT), jnp.float32),
        in_specs=[
            pl.BlockSpec(memory_space=pltpu.VMEM),
            pl.BlockSpec(memory_space=pltpu.VMEM),
            pl.BlockSpec(memory_space=pltpu.VMEM),
            pl.BlockSpec(memory_space=pltpu.VMEM),
        ],
        out_specs=pl.BlockSpec(memory_space=pltpu.VMEM),
        scratch_shapes=[
            pltpu.VMEM((N_DEV, PAY_ROWS, D_OUT), jnp.float32),
            pltpu.SemaphoreType.DMA((R_HOPS, 2)),
            pltpu.SemaphoreType.DMA((R_HOPS, 2)),
            pltpu.SemaphoreType.DMA((L_HOPS, 2)),
            pltpu.SemaphoreType.DMA((L_HOPS, 2)),
        ],
        compiler_params=pltpu.CompilerParams(collective_id=0),
    )(payload, x, route_idx, local_rank)


# device time: 44052 ns/iter; 1.9095x vs baseline; 1.0208x over previous
import jax
import jax.numpy as jnp
from jax import lax
from jax.experimental import pallas as pl
from jax.experimental.pallas import tpu as pltpu

N_DEV = 16
N_EXPERTS = 32
CAPACITY = 102
N_TOK = 256
D_IN = 128
D_OUT = 256
E_PER = 2
W_ROWS = E_PER * D_IN
PAY_ROWS = W_ROWS + 8


def kernel(x, router_W, route_idx, expert_W):
    del router_W

    onehot = (route_idx == jnp.arange(N_EXPERTS, dtype=jnp.int32)[None, :])
    cums = jnp.cumsum(onehot.astype(jnp.int32), axis=0)
    local_rank = jnp.take_along_axis(cums, route_idx, axis=1) - 1
    counts_local = cums[-1]

    w_flat = expert_W.reshape(W_ROWS, D_OUT)
    pad = jnp.zeros((8, D_OUT), jnp.int32)
    pad = pad.at[0, :N_EXPERTS].set(counts_local)
    payload = jnp.concatenate(
        [w_flat, lax.bitcast_convert_type(pad, jnp.float32)], axis=0
    )

    R_HOPS = N_DEV // 2
    L_HOPS = N_DEV - 1 - R_HOPS

    def body(payload_ref, x_ref, ri_ref, rank_ref, out_ref,
             comm_ref, send_r, recv_r, send_l, recv_l):
        my = lax.axis_index("i")
        left = lax.rem(my - 1 + N_DEV, N_DEV)
        right = lax.rem(my + 1, N_DEV)

        barrier_sem = pltpu.get_barrier_semaphore()
        for nbr in (left, right):
            pl.semaphore_signal(
                barrier_sem, inc=1,
                device_id=(nbr,), device_id_type=pl.DeviceIdType.MESH,
            )
        pl.semaphore_wait(barrier_sem, 2)

        comm_ref[0] = payload_ref[:, :]

        xv = x_ref[:, :]
        ri = ri_ref[:, :]

        def process(slot, src, base):
            e0 = E_PER * src
            m0 = (ri == e0).astype(jnp.float32)
            m1 = (ri == e0 + 1).astype(jnp.float32)
            w0 = comm_ref[slot, 0:D_IN, :]
            w1 = comm_ref[slot, D_IN:W_ROWS, :]
            contrib = (
                jnp.dot(xv * m0, w0, preferred_element_type=jnp.float32)
                + jnp.dot(xv * m1, w1, preferred_element_type=jnp.float32)
            )
            if slot == 0:
                out_ref[:, :] = contrib
            else:
                out_ref[:, :] = out_ref[:, :] + contrib
            counts_k = lax.bitcast_convert_type(
                comm_ref[slot, W_ROWS:W_ROWS + 1, :], jnp.int32
            )
            return base + jnp.where(src < my, counts_k, 0)

        SUBS = ((0, 64), (64, 64), (128, 64), (192, PAY_ROWS - 192))

        def send_sub(src_slot, dst_slot, sem_arr, j, rsem_arr, nbr, c):
            off, sz = SUBS[c]
            rdma = pltpu.make_async_remote_copy(
                src_ref=comm_ref.at[src_slot, pl.ds(off, sz)],
                dst_ref=comm_ref.at[dst_slot, pl.ds(off, sz)],
                send_sem=sem_arr.at[j, c],
                recv_sem=rsem_arr.at[j, c],
                device_id=(nbr,),
                device_id_type=pl.DeviceIdType.MESH,
            )
            rdma.start()
            return rdma

        def send(src_slot, dst_slot, sem_arr, j, rsem_arr, nbr):
            return [
                send_sub(src_slot, dst_slot, sem_arr, j, rsem_arr, nbr, c)
                for c in range(len(SUBS))
            ]

        def r_slot(j):
            return j

        def l_slot(j):
            return R_HOPS + j

        all_rdmas = []
        r_prev = send(0, r_slot(1), send_r, 0, recv_r, right)
        l_prev = send(0, l_slot(1), send_l, 0, recv_l, left)
        all_rdmas += r_prev + l_prev

        base = jnp.zeros((1, D_OUT), jnp.int32)
        base = process(0, my, base)

        for j in range(1, R_HOPS + 1):
            r_next, l_next = [], []
            for c in range(len(SUBS)):
                r_prev[c].wait_recv()
                if j < R_HOPS:
                    r_next.append(
                        send_sub(r_slot(j), r_slot(j + 1),
                                 send_r, j, recv_r, right, c)
                    )
                if j <= L_HOPS:
                    l_prev[c].wait_recv()
                    if j < L_HOPS:
                        l_next.append(
                            send_sub(l_slot(j), l_slot(j + 1),
                                     send_l, j, recv_l, left, c)
                        )
            all_rdmas += r_next + l_next
            base = process(r_slot(j), lax.rem(my - j + N_DEV, N_DEV), base)
            if j <= L_HOPS:
                base = process(l_slot(j), lax.rem(my + j, N_DEV), base)
            r_prev, l_prev = r_next, l_next

        for rdma in all_rdmas:
            rdma.wait_send()

        col = lax.broadcasted_iota(jnp.int32, (N_TOK, D_OUT), 1)
        oh = (ri == col)
        base_i = jnp.sum(
            jnp.where(oh, jnp.broadcast_to(base, (N_TOK, D_OUT)), 0),
            axis=1, keepdims=True,
        )
        keep = ((base_i + rank_ref[:, :]) < CAPACITY).astype(jnp.float32)
        out_ref[:, :] = out_ref[:, :] * keep

    return pl.pallas_call(
        body,
        out_shape=jax.ShapeDtypeStruct((N_TOK, D_OUT), jnp.float32),
        in_specs=[
            pl.BlockSpec(memory_space=pltpu.VMEM),
            pl.BlockSpec(memory_space=pltpu.VMEM),
            pl.BlockSpec(memory_space=pltpu.VMEM),
            pl.BlockSpec(memory_space=pltpu.VMEM),
        ],
        out_specs=pl.BlockSpec(memory_space=pltpu.VMEM),
        scratch_shapes=[
            pltpu.VMEM((N_DEV, PAY_ROWS, D_OUT), jnp.float32),
            pltpu.SemaphoreType.DMA((R_HOPS, 4)),
            pltpu.SemaphoreType.DMA((R_HOPS, 4)),
            pltpu.SemaphoreType.DMA((L_HOPS, 4)),
            pltpu.SemaphoreType.DMA((L_HOPS, 4)),
        ],
        compiler_params=pltpu.CompilerParams(collective_id=0),
    )(payload, x, route_idx, local_rank)


# device time: 35505 ns/iter; 2.3692x vs baseline; 1.2407x over previous
import jax
import jax.numpy as jnp
from jax import lax
from jax.experimental import pallas as pl
from jax.experimental.pallas import tpu as pltpu

N_DEV = 16
N_EXPERTS = 32
CAPACITY = 102
N_TOK = 256
D_IN = 128
D_OUT = 256
E_PER = 2
W_ROWS = E_PER * D_IN
PAY_ROWS = W_ROWS + 16


def kernel(x, router_W, route_idx, expert_W):
    del router_W

    onehot = (route_idx == jnp.arange(N_EXPERTS, dtype=jnp.int32)[None, :])
    counts_local = jnp.sum(onehot, axis=0, dtype=jnp.int32)

    w_flat = expert_W.reshape(W_ROWS, D_OUT).astype(jnp.bfloat16)
    pad = jnp.zeros((PAY_ROWS - W_ROWS, D_OUT), jnp.bfloat16)
    pad = pad.at[0, :N_EXPERTS].set(counts_local.astype(jnp.bfloat16))
    payload = jnp.concatenate([w_flat, pad], axis=0)

    R_HOPS = N_DEV // 2
    L_HOPS = N_DEV - 1 - R_HOPS

    def body(payload_ref, x_ref, ri_ref, out_ref,
             comm_ref, send_r, recv_r, send_l, recv_l):
        my = lax.axis_index("i")
        left = lax.rem(my - 1 + N_DEV, N_DEV)
        right = lax.rem(my + 1, N_DEV)

        barrier_sem = pltpu.get_barrier_semaphore()
        for nbr in (left, right):
            pl.semaphore_signal(
                barrier_sem, inc=1,
                device_id=(nbr,), device_id_type=pl.DeviceIdType.MESH,
            )
        pl.semaphore_wait(barrier_sem, 2)

        xv = x_ref[:, :].astype(jnp.bfloat16)
        ri = ri_ref[:, :]

        def process(chunk, src, base, first=False):
            e0 = E_PER * src
            m0 = (ri == e0).astype(jnp.bfloat16)
            m1 = (ri == e0 + 1).astype(jnp.bfloat16)
            w0 = chunk[0:D_IN, :]
            w1 = chunk[D_IN:W_ROWS, :]
            contrib = (
                jnp.dot(xv * m0, w0, preferred_element_type=jnp.float32)
                + jnp.dot(xv * m1, w1, preferred_element_type=jnp.float32)
            )
            if first:
                out_ref[:, :] = contrib
            else:
                out_ref[:, :] = out_ref[:, :] + contrib
            counts_k = chunk[W_ROWS:W_ROWS + 1, :].astype(jnp.int32)
            return base + jnp.where(src < my, counts_k, 0)

        SUBS = ((0, 64), (64, 64), (128, 64), (192, PAY_ROWS - 192))

        def send_sub(src_ref, dst_slot, sem_arr, j, rsem_arr, nbr, c):
            off, sz = SUBS[c]
            rdma = pltpu.make_async_remote_copy(
                src_ref=src_ref.at[pl.ds(off, sz)],
                dst_ref=comm_ref.at[dst_slot, pl.ds(off, sz)],
                send_sem=sem_arr.at[j, c],
                recv_sem=rsem_arr.at[j, c],
                device_id=(nbr,),
                device_id_type=pl.DeviceIdType.MESH,
            )
            rdma.start()
            return rdma

        def send(src_ref, dst_slot, sem_arr, j, rsem_arr, nbr):
            return [
                send_sub(src_ref, dst_slot, sem_arr, j, rsem_arr, nbr, c)
                for c in range(len(SUBS))
            ]

        def r_slot(j):
            return j

        def l_slot(j):
            return R_HOPS + j

        all_rdmas = []
        r_prev = send(payload_ref, r_slot(1), send_r, 0, recv_r, right)
        l_prev = send(payload_ref, l_slot(1), send_l, 0, recv_l, left)
        all_rdmas += r_prev + l_prev

        base = jnp.zeros((1, D_OUT), jnp.int32)
        base = process(payload_ref, my, base, first=True)

        r_row = lax.broadcasted_iota(jnp.int32, (N_TOK, N_TOK), 0)
        r_col = lax.broadcasted_iota(jnp.int32, (N_TOK, N_TOK), 1)
        tri = (r_col < r_row).astype(jnp.bfloat16)
        ohq = (ri == lax.broadcasted_iota(
            jnp.int32, (N_TOK, N_EXPERTS), 1)).astype(jnp.bfloat16)
        cums_strict = jnp.dot(tri, ohq, preferred_element_type=jnp.float32)
        rank = jnp.sum(
            ohq.astype(jnp.float32) * cums_strict, axis=1, keepdims=True
        ).astype(jnp.int32)

        for j in range(1, R_HOPS + 1):
            r_next, l_next = [], []
            for c in range(len(SUBS)):
                r_prev[c].wait_recv()
                if j < R_HOPS:
                    r_next.append(
                        send_sub(comm_ref.at[r_slot(j)], r_slot(j + 1),
                                 send_r, j, recv_r, right, c)
                    )
                if j <= L_HOPS:
                    l_prev[c].wait_recv()
                    if j < L_HOPS:
                        l_next.append(
                            send_sub(comm_ref.at[l_slot(j)], l_slot(j + 1),
                                     send_l, j, recv_l, left, c)
                        )
            all_rdmas += r_next + l_next
            base = process(comm_ref.at[r_slot(j)],
                           lax.rem(my - j + N_DEV, N_DEV), base)
            if j <= L_HOPS:
                base = process(comm_ref.at[l_slot(j)],
                               lax.rem(my + j, N_DEV), base)
            r_prev, l_prev = r_next, l_next

        for rdma in all_rdmas:
            rdma.wait_send()

        col = lax.broadcasted_iota(jnp.int32, (N_TOK, D_OUT), 1)
        oh = (ri == col)
        base_i = jnp.sum(
            jnp.where(oh, jnp.broadcast_to(base, (N_TOK, D_OUT)), 0),
            axis=1, keepdims=True,
        )
        keep = ((base_i + rank) < CAPACITY).astype(jnp.float32)
        out_ref[:, :] = out_ref[:, :] * keep

    return pl.pallas_call(
        body,
        out_shape=jax.ShapeDtypeStruct((N_TOK, D_OUT), jnp.float32),
        in_specs=[
            pl.BlockSpec(memory_space=pltpu.VMEM),
            pl.BlockSpec(memory_space=pltpu.VMEM),
            pl.BlockSpec(memory_space=pltpu.VMEM),
        ],
        out_specs=pl.BlockSpec(memory_space=pltpu.VMEM),
        scratch_shapes=[
            pltpu.VMEM((N_DEV, PAY_ROWS, D_OUT), jnp.bfloat16),
            pltpu.SemaphoreType.DMA((R_HOPS, 4)),
            pltpu.SemaphoreType.DMA((R_HOPS, 4)),
            pltpu.SemaphoreType.DMA((L_HOPS, 4)),
            pltpu.SemaphoreType.DMA((L_HOPS, 4)),
        ],
        compiler_params=pltpu.CompilerParams(collective_id=0),
    )(payload, x, route_idx)
